# Initial kernel scaffold; baseline (speedup 1.0000x reference)
import jax
import jax.numpy as jnp
from jax.experimental import pallas as pl


def kernel(x):
    raise NotImplementedError("write your pallas kernel here")


# speedup vs baseline: 1.0292x; 1.0292x over previous
import jax
import jax.numpy as jnp
from jax.experimental import pallas as pl


def _copy_block(x_ref, o_ref):
    o_ref[...] = x_ref[...]


def kernel(x):
    rows, cols = x.shape
    block_rows = 1024
    grid = (rows // block_rows,)
    gathered = pl.pallas_call(
        _copy_block,
        grid=grid,
        in_specs=[pl.BlockSpec((block_rows, cols), lambda i: (i, 0))],
        out_specs=pl.BlockSpec((block_rows, cols), lambda i: (i, 0)),
        out_shape=jax.ShapeDtypeStruct((rows, cols), x.dtype),
    )(x)
    sizes = jnp.array([rows], dtype=jnp.int64)
    return (gathered, sizes)
